# baseline (speedup 1.0000x reference)
import functools

import jax
import jax.numpy as jnp
from jax import lax
from jax.experimental import pallas as pl
from jax.experimental.pallas import tpu as pltpu
from jax.experimental.pallas import tpu_sc as plsc

NC = 2
NS = 16
NW = NC * NS

G = 128
GROUPS = 10
CHUNK = G * GROUPS


def _gather_kernel(hidden, n_pairs, idx_hbm, table_hbm, out_hbm,
                   idx0, idx1, rows0, rows1, gsem0, gsem1, wsem0, wsem1):
    wid = lax.axis_index("s") * NC + lax.axis_index("c")
    n_chunks = 2 * n_pairs
    group_base = wid * (n_chunks * GROUPS)
    row_base = wid * (n_chunks * CHUNK)

    bufs = ((idx0, rows0, gsem0, wsem0), (idx1, rows1, gsem1, wsem1))

    def issue(c, sub):
        idx_v, rows_v, gsem, _ = bufs[sub]
        pltpu.sync_copy(idx_hbm.at[pl.ds(group_base + c * GROUPS, GROUPS)],
                        idx_v)
        for j in range(GROUPS):
            pltpu.async_copy(table_hbm.at[idx_v.at[j]],
                             rows_v.at[pl.ds(j * G, G)], gsem)

    def wait_gathers(sub):
        idx_v, rows_v, gsem, _ = bufs[sub]
        for j in range(GROUPS):
            pltpu.make_async_copy(table_hbm.at[idx_v.at[j]],
                                  rows_v.at[pl.ds(j * G, G)], gsem).wait()

    def write_back(c, sub):
        _, rows_v, _, wsem = bufs[sub]
        pltpu.async_copy(rows_v,
                         out_hbm.at[pl.ds(row_base + c * CHUNK, CHUNK)], wsem)

    def wait_write(sub):
        _, rows_v, _, wsem = bufs[sub]
        pltpu.make_async_copy(rows_v, out_hbm.at[pl.ds(row_base, CHUNK)],
                              wsem).wait()

    def drain(c, sub):
        wait_gathers(sub)
        write_back(c, sub)
        wait_write(sub)

    issue(0, 0)
    issue(1, 1)

    def body(k, _):
        drain(2 * k, 0)
        issue(2 * k + 2, 0)
        drain(2 * k + 1, 1)
        issue(2 * k + 3, 1)
        return ()

    lax.fori_loop(0, n_pairs - 1, body, (), unroll=False)

    drain(n_chunks - 2, 0)
    drain(n_chunks - 1, 1)


def kernel(input_ids, position_ids, word_embeddings):
    batch, seq = input_ids.shape
    vocab, hidden = word_embeddings.shape
    n = batch * seq
    assert n % (NW * 2 * CHUNK) == 0
    n_pairs = n // (NW * 2 * CHUNK)

    idx_flat = input_ids.reshape(n // G, G)

    mesh = plsc.VectorSubcoreMesh(core_axis_name="c", subcore_axis_name="s")
    gather = pl.kernel(
        functools.partial(_gather_kernel, hidden, n_pairs),
        out_type=jax.ShapeDtypeStruct((n, hidden), jnp.float32),
        mesh=mesh,
        scratch_types=[
            pltpu.VMEM((GROUPS, G), jnp.int32),
            pltpu.VMEM((GROUPS, G), jnp.int32),
            pltpu.VMEM((CHUNK, hidden), jnp.float32),
            pltpu.VMEM((CHUNK, hidden), jnp.float32),
            pltpu.SemaphoreType.DMA,
            pltpu.SemaphoreType.DMA,
            pltpu.SemaphoreType.DMA,
            pltpu.SemaphoreType.DMA,
        ],
        compiler_params=pltpu.CompilerParams(use_tc_tiling_on_sc=False),
    )
    out = gather(idx_flat, word_embeddings)
    return (out.reshape(batch, seq, hidden), position_ids)


# speedup vs baseline: 1.0007x; 1.0007x over previous
import functools

import jax
import jax.numpy as jnp
from jax import lax
from jax.experimental import pallas as pl
from jax.experimental.pallas import tpu as pltpu
from jax.experimental.pallas import tpu_sc as plsc

NC = 2
NS = 16
NW = NC * NS

G = 128
GROUPS = 10
CHUNK = G * GROUPS


def _gather_kernel(hidden, n_pairs, idx_hbm, table_hbm, out_hbm,
                   idx0, idx1, rows0, rows1, gsem0, gsem1, wsem0, wsem1):
    wid = lax.axis_index("s") * NC + lax.axis_index("c")
    n_chunks = 2 * n_pairs
    row_base = wid * (n_chunks * CHUNK)

    bufs = ((idx0, rows0, gsem0, wsem0), (idx1, rows1, gsem1, wsem1))

    def issue(c, sub):
        idx_v, rows_v, gsem, _ = bufs[sub]
        pltpu.sync_copy(idx_hbm.at[pl.ds(row_base + c * CHUNK, CHUNK)],
                        idx_v)
        pltpu.async_copy(table_hbm.at[idx_v], rows_v, gsem)

    def drain(c, sub):
        idx_v, rows_v, gsem, wsem = bufs[sub]
        pltpu.make_async_copy(table_hbm.at[idx_v], rows_v, gsem).wait()
        dst = out_hbm.at[pl.ds(row_base + c * CHUNK, CHUNK)]
        pltpu.async_copy(rows_v, dst, wsem)
        pltpu.make_async_copy(rows_v, dst, wsem).wait()

    issue(0, 0)
    issue(1, 1)

    def body(k, _):
        drain(2 * k, 0)
        issue(2 * k + 2, 0)
        drain(2 * k + 1, 1)
        issue(2 * k + 3, 1)
        return ()

    lax.fori_loop(0, n_pairs - 1, body, (), unroll=False)

    drain(n_chunks - 2, 0)
    drain(n_chunks - 1, 1)


def kernel(input_ids, position_ids, word_embeddings):
    batch, seq = input_ids.shape
    vocab, hidden = word_embeddings.shape
    n = batch * seq
    assert n % (NW * 2 * CHUNK) == 0
    n_pairs = n // (NW * 2 * CHUNK)

    idx_flat = input_ids.reshape(n)

    mesh = plsc.VectorSubcoreMesh(core_axis_name="c", subcore_axis_name="s")
    gather = pl.kernel(
        functools.partial(_gather_kernel, hidden, n_pairs),
        out_type=jax.ShapeDtypeStruct((n, hidden), jnp.float32),
        mesh=mesh,
        scratch_types=[
            pltpu.VMEM((CHUNK,), jnp.int32),
            pltpu.VMEM((CHUNK,), jnp.int32),
            pltpu.VMEM((CHUNK, hidden), jnp.float32),
            pltpu.VMEM((CHUNK, hidden), jnp.float32),
            pltpu.SemaphoreType.DMA,
            pltpu.SemaphoreType.DMA,
            pltpu.SemaphoreType.DMA,
            pltpu.SemaphoreType.DMA,
        ],
        compiler_params=pltpu.CompilerParams(use_tc_tiling_on_sc=False),
    )
    out = gather(idx_flat, word_embeddings)
    return (out.reshape(batch, seq, hidden), position_ids)


# speedup vs baseline: 1.7372x; 1.7361x over previous
import functools

import jax
import jax.numpy as jnp
from jax import lax
from jax.experimental import pallas as pl
from jax.experimental.pallas import tpu as pltpu
from jax.experimental.pallas import tpu_sc as plsc

NC = 2
NS = 16
NW = NC * NS

G = 128
GROUPS = 10
CHUNK = G * GROUPS


def _gather_kernel(hidden, n_pairs, idx_hbm, table_hbm, out_hbm,
                   idx0, idx1, rows0, rows1, gsem0, gsem1, wsem0, wsem1):
    wid = lax.axis_index("s") * NC + lax.axis_index("c")
    n_chunks = 2 * n_pairs
    row_base = wid * (n_chunks * CHUNK)

    bufs = ((idx0, rows0, gsem0, wsem0), (idx1, rows1, gsem1, wsem1))

    def issue(c, sub):
        idx_v, rows_v, gsem, _ = bufs[sub]
        pltpu.sync_copy(idx_hbm.at[pl.ds(row_base + c * CHUNK, CHUNK)],
                        idx_v)
        pltpu.async_copy(table_hbm.at[idx_v], rows_v, gsem)

    def drain(c, sub):
        idx_v, rows_v, gsem, wsem = bufs[sub]
        pltpu.make_async_copy(table_hbm.at[idx_v], rows_v, gsem).wait()
        dst = out_hbm.at[pl.ds(row_base + c * CHUNK, CHUNK)]
        pltpu.async_copy(rows_v, dst, wsem)
        pltpu.make_async_copy(rows_v, dst, wsem).wait()

    issue(0, 0)
    issue(1, 1)

    def body(k, _):
        drain(2 * k, 0)
        issue(2 * k + 2, 0)
        drain(2 * k + 1, 1)
        issue(2 * k + 3, 1)
        return ()

    lax.fori_loop(0, n_pairs - 1, body, (), unroll=False)

    drain(n_chunks - 2, 0)
    drain(n_chunks - 1, 1)


def kernel(input_ids, position_ids, word_embeddings):
    batch, seq = input_ids.shape
    vocab, hidden = word_embeddings.shape
    n = batch * seq
    assert n % (NW * 2 * CHUNK) == 0
    n_pairs = n // (NW * 2 * CHUNK)

    idx_flat = input_ids.T.reshape(n)

    mesh = plsc.VectorSubcoreMesh(core_axis_name="c", subcore_axis_name="s")
    gather = pl.kernel(
        functools.partial(_gather_kernel, hidden, n_pairs),
        out_type=jax.ShapeDtypeStruct((n, hidden), jnp.float32),
        mesh=mesh,
        scratch_types=[
            pltpu.VMEM((CHUNK,), jnp.int32),
            pltpu.VMEM((CHUNK,), jnp.int32),
            pltpu.VMEM((CHUNK, hidden), jnp.float32),
            pltpu.VMEM((CHUNK, hidden), jnp.float32),
            pltpu.SemaphoreType.DMA,
            pltpu.SemaphoreType.DMA,
            pltpu.SemaphoreType.DMA,
            pltpu.SemaphoreType.DMA,
        ],
        compiler_params=pltpu.CompilerParams(use_tc_tiling_on_sc=False),
    )
    out = gather(idx_flat, word_embeddings)
    return (out.reshape(seq, batch, hidden).transpose(1, 0, 2), position_ids)
